# Initial kernel scaffold; baseline (speedup 1.0000x reference)
import jax
import jax.numpy as jnp
from jax.experimental import pallas as pl


def kernel(x, edge_index, batch, metadata, W1, b1, W2, b2, W3, b3, W4, b4, p1, p2, p3, p4, convm_w, convm_b, fc_w, fc_b, fc2_w, fc2_b):
    raise NotImplementedError("write your pallas kernel here")


# speedup vs baseline: 15.9839x; 15.9839x over previous
import functools
import math

import jax
import jax.numpy as jnp
from jax import lax
from jax.experimental import pallas as pl
from jax.experimental.pallas import tpu as pltpu
from jax.experimental.pallas import tpu_sc as plsc

N = 10000
E = 320000
NPAD = 10240
F = 128
R = 1000
NEG_INF = float("-inf")

_SC_NC = 2
_SC_NS = 16
_NW = _SC_NC * _SC_NS
_EP = E // _NW
_CHUNK = 80
_NACC = NPAD
_RPT = _NACC // _SC_NS


@functools.lru_cache(maxsize=None)
def _make_edge_pass(D):
    mesh = plsc.VectorSubcoreMesh(core_axis_name="c", subcore_axis_name="s",
                                  num_cores=_SC_NC, num_subcores=_SC_NS)

    @functools.partial(
        pl.kernel,
        out_type=jax.ShapeDtypeStruct((2 * _NACC, D), jnp.float32),
        mesh=mesh,
        scratch_types=[
            pltpu.VMEM((_CHUNK,), jnp.int32),
            pltpu.VMEM((_CHUNK,), jnp.int32),
            pltpu.VMEM((_CHUNK, D), jnp.float32),
            pltpu.VMEM_SHARED((_NACC, D), jnp.float32),
            pltpu.SemaphoreType.DMA,
        ],
    )
    def edge_pass(src_hbm, dst_hbm, tab_hbm, zeros_hbm, out_hbm,
                  src_v, dst_v, rows_v, z_sh, sem):
        c = lax.axis_index("c")
        s = lax.axis_index("s")
        w = c * _SC_NS + s
        rbase = s * _RPT
        pltpu.sync_copy(zeros_hbm, z_sh.at[pl.ds(rbase, _RPT)])
        plsc.subcore_barrier()
        ebase = w * _EP

        def body(i, carry):
            off = ebase + i * _CHUNK
            pltpu.sync_copy(src_hbm.at[pl.ds(off, _CHUNK)], src_v)
            pltpu.sync_copy(dst_hbm.at[pl.ds(off, _CHUNK)], dst_v)
            pltpu.async_copy(tab_hbm.at[src_v], rows_v, sem).wait()
            pltpu.sync_copy(rows_v, z_sh.at[dst_v], add=True)
            return carry

        lax.fori_loop(0, _EP // _CHUNK, body, 0)
        plsc.subcore_barrier()
        pltpu.sync_copy(z_sh.at[pl.ds(rbase, _RPT)],
                        out_hbm.at[pl.ds(c * _NACC + rbase, _RPT)])

    return edge_pass


def _edge_pass128(src, dst, tab, zeros):
    return _make_edge_pass(F)(src, dst, tab, zeros)


@functools.lru_cache(maxsize=None)
def _make_s0_pass():
    mesh = plsc.VectorSubcoreMesh(core_axis_name="c", subcore_axis_name="s",
                                  num_cores=_SC_NC, num_subcores=_SC_NS)
    n_chunks = _EP // _CHUNK

    @functools.partial(
        pl.kernel,
        out_type=jax.ShapeDtypeStruct((2 * NPAD,), jnp.float32),
        mesh=mesh,
        scratch_types=[
            pltpu.VMEM((_CHUNK,), jnp.int32),
            pltpu.VMEM((_CHUNK,), jnp.int32),
            pltpu.VMEM((_CHUNK,), jnp.float32),
            pltpu.VMEM_SHARED((NPAD,), jnp.float32),
            pltpu.SemaphoreType.DMA,
        ],
    )
    def s0_pass(src_hbm, dst_hbm, m_hbm, zeros_hbm, out_hbm,
                src_v, dst_v, vals_v, z_sh, sem):
        c = lax.axis_index("c")
        s = lax.axis_index("s")
        w = c * _SC_NS + s
        rbase = s * (NPAD // _SC_NS)
        pltpu.sync_copy(zeros_hbm, z_sh.at[pl.ds(rbase, NPAD // _SC_NS)])
        plsc.subcore_barrier()
        ebase = w * _EP

        def body(i, carry):
            off = ebase + i * _CHUNK
            pltpu.sync_copy(src_hbm.at[pl.ds(off, _CHUNK)], src_v)
            pltpu.sync_copy(dst_hbm.at[pl.ds(off, _CHUNK)], dst_v)
            pltpu.async_copy(m_hbm.at[src_v], vals_v, sem).wait()
            pltpu.sync_copy(vals_v, z_sh.at[dst_v], add=True)
            return carry

        lax.fori_loop(0, n_chunks, body, 0)
        plsc.subcore_barrier()
        pltpu.sync_copy(z_sh.at[pl.ds(rbase, NPAD // _SC_NS)],
                        out_hbm.at[pl.ds(c * NPAD + rbase, NPAD // _SC_NS)])

    return s0_pass


def _s0_pass(src, dst, m_pad, zeros_pad):
    return _make_s0_pass()(src, dst, m_pad, zeros_pad)


def _row_spec(d):
    return pl.BlockSpec((R, d), lambda i: (i, 0))


def _full_spec(shape):
    return pl.BlockSpec(shape, lambda i: tuple(0 for _ in shape))


def _pre1_body(x8_ref, s0a_ref, s0b_ref, w1_ref, hs_ref, dm_ref):
    deg = s0a_ref[...] + s0b_ref[...] + 1.0
    dinv = lax.rsqrt(deg)
    dm_ref[...] = dinv
    h = jnp.dot(x8_ref[...], w1_ref[...], preferred_element_type=jnp.float32)
    hs_ref[...] = h * dinv


_pre1 = pl.pallas_call(
    _pre1_body,
    grid=(N // R,),
    in_specs=[_row_spec(8), _row_spec(1), _row_spec(1), _full_spec((8, F))],
    out_specs=[_row_spec(F), _row_spec(1)],
    out_shape=[jax.ShapeDtypeStruct((N, F), jnp.float32),
               jax.ShapeDtypeStruct((N, 1), jnp.float32)],
)


def _pre_body(prev_ref, ps_ref, m_ref, s0a_ref, s0b_ref, w_ref,
              hs_ref, dm_ref):
    m = m_ref[...]
    xcur = prev_ref[...] * jnp.tanh(ps_ref[...]) * m
    h = jnp.dot(xcur, w_ref[...], preferred_element_type=jnp.float32)
    deg = m * (s0a_ref[...] + s0b_ref[...]) + 1.0
    dm = lax.rsqrt(deg) * m
    dm_ref[...] = dm
    hs_ref[...] = h * dm


_pre = pl.pallas_call(
    _pre_body,
    grid=(N // R,),
    in_specs=[_row_spec(F), _row_spec(1), _row_spec(1), _row_spec(1),
              _row_spec(1), _full_spec((F, F))],
    out_specs=[_row_spec(F), _row_spec(1)],
    out_shape=[jax.ShapeDtypeStruct((N, F), jnp.float32),
               jax.ShapeDtypeStruct((N, 1), jnp.float32)],
)


def _post_body(sa_ref, sb_ref, hs_ref, dm_ref, m_ref, b_ref, p_ref,
               out_ref, sc_ref):
    z = (sa_ref[...] + sb_ref[...] + hs_ref[...]) * dm_ref[...]
    out = jnp.maximum(z + b_ref[...], 0.0) * m_ref[...]
    out_ref[...] = out
    p = p_ref[...]
    pnorm = jnp.sqrt(jnp.sum(p * p))
    sc_ref[...] = lax.dot_general(
        out, p, (((1,), (1,)), ((), ())),
        preferred_element_type=jnp.float32) / pnorm


_post = pl.pallas_call(
    _post_body,
    grid=(N // R,),
    in_specs=[_row_spec(F), _row_spec(F), _row_spec(F), _row_spec(1),
              _row_spec(1), _full_spec((1, F)), _full_spec((1, F))],
    out_specs=[_row_spec(F), _row_spec(1)],
    out_shape=[jax.ShapeDtypeStruct((N, F), jnp.float32),
               jax.ShapeDtypeStruct((N, 1), jnp.float32)],
)


def _make_topk(k):
    def body(s_ref, m_ref, sel_ref):
        s = s_ref[...]
        m = m_ref[...]
        bits = lax.bitcast_convert_type(s, jnp.uint32)
        sign = bits >> jnp.uint32(31)
        flip = (jnp.uint32(0x80000000)
                | (jnp.uint32(0) - sign))
        key = jnp.where(m > 0, bits ^ flip, jnp.uint32(0))

        def kbit(i, kacc):
            sh = (jnp.uint32(31) - i.astype(jnp.uint32))
            cand = kacc | (jnp.uint32(1) << sh)
            cnt = jnp.sum((key >= cand).astype(jnp.int32))
            return jnp.where(cnt >= k, cand, kacc)

        kth = lax.fori_loop(0, 32, kbit, jnp.uint32(0))
        gt = key > kth
        need = k - jnp.sum(gt.astype(jnp.int32))
        tie = key == kth
        idx = (lax.broadcasted_iota(jnp.int32, (80, 128), 0) * 128
               + lax.broadcasted_iota(jnp.int32, (80, 128), 1))

        def mbit(i, macc):
            cand = macc | (jnp.int32(1) << (jnp.int32(14) - i))
            cnt = jnp.sum((tie & (idx < cand)).astype(jnp.int32))
            return jnp.where(cnt < need, cand, macc)

        mth = lax.fori_loop(0, 15, mbit, jnp.int32(0))
        sel = gt | (tie & (idx <= mth) & (need > 0))
        sel_ref[...] = sel.astype(jnp.float32)

    return pl.pallas_call(
        body,
        in_specs=[pl.BlockSpec((80, 128), lambda: (0, 0)),
                  pl.BlockSpec((80, 128), lambda: (0, 0))],
        out_specs=pl.BlockSpec((80, 128), lambda: (0, 0)),
        out_shape=jax.ShapeDtypeStruct((80, 128), jnp.float32),
    )


_topk_kernels = {k: _make_topk(k) for k in (5000, 2500, 1250, 625)}


def _head_body(out4_ref, s4_ref, m4_ref, meta_ref, cw_ref, cb_ref,
               fcw_ref, fcb_ref, fc2w_ref, fc2b_ref, o_ref, smax, ssum):
    pid = pl.program_id(0)

    @pl.when(pid == 0)
    def _init():
        smax[...] = jnp.full((8, 128), NEG_INF, jnp.float32)
        ssum[...] = jnp.zeros((8, 128), jnp.float32)

    m4 = m4_ref[...]
    x4 = out4_ref[...] * jnp.tanh(s4_ref[...]) * m4
    bmax = jnp.max(jnp.where(m4 > 0, x4, NEG_INF), axis=0, keepdims=True)
    bsum = jnp.sum(x4, axis=0, keepdims=True)
    smax[0:1, :] = jnp.maximum(smax[0:1, :], bmax)
    ssum[0:1, :] = ssum[0:1, :] + bsum
    o_ref[...] = jnp.zeros((1, 1), jnp.float32)

    @pl.when(pid == (N // R) - 1)
    def _fin():
        xmax = smax[0:1, :]
        xmean = ssum[0:1, :] * (1.0 / 625.0)
        mr = jnp.maximum(meta_ref[...] * cw_ref[...] + cb_ref[...], 0.0)
        fcw = fcw_ref[...]
        h1 = (lax.dot_general(xmax, fcw[:, 0:F], (((1,), (1,)), ((), ())),
                              preferred_element_type=jnp.float32)
              + lax.dot_general(xmean, fcw[:, F:2 * F],
                                (((1,), (1,)), ((), ())),
                                preferred_element_type=jnp.float32)
              + lax.dot_general(mr[:, 0:4], fcw[:, 2 * F:2 * F + 4],
                                (((1,), (1,)), ((), ())),
                                preferred_element_type=jnp.float32)
              + fcb_ref[...])
        h1 = jnp.maximum(h1, 0.0)
        o_ref[...] = (jnp.sum(h1 * fc2w_ref[...], axis=1, keepdims=True)
                      + fc2b_ref[...])


_head = pl.pallas_call(
    _head_body,
    grid=(N // R,),
    in_specs=[_row_spec(F), _row_spec(1), _row_spec(1),
              _full_spec((1, F)), _full_spec((1, F)), _full_spec((1, F)),
              _full_spec((F, 2 * F + 4)), _full_spec((1, F)),
              _full_spec((1, F)), _full_spec((1, 1))],
    out_specs=pl.BlockSpec((1, 1), lambda i: (0, 0)),
    out_shape=jax.ShapeDtypeStruct((1, 1), jnp.float32),
    scratch_shapes=[pltpu.VMEM((8, 128), jnp.float32),
                    pltpu.VMEM((8, 128), jnp.float32)],
)


def _pad_scores(s_col, m_col):
    spad = jnp.pad(s_col[:, 0], (0, NPAD - N),
                   constant_values=NEG_INF).reshape(80, 128)
    mpad = jnp.pad(m_col[:, 0], (0, NPAD - N),
                   constant_values=0.0).reshape(80, 128)
    return spad, mpad


def kernel(x, edge_index, batch, metadata, W1, b1, W2, b2, W3, b3, W4, b4,
           p1, p2, p3, p4, convm_w, convm_b, fc_w, fc_b, fc2_w, fc2_b):
    f32 = jnp.float32
    src = edge_index[0]
    dst = edge_index[1]
    zeros128 = jnp.zeros((_RPT, F), f32)
    ones2d = jnp.ones((80, 128), f32)

    zeros_s0 = jnp.zeros((NPAD // _SC_NS,), f32)

    def s0_cols(m2d):
        s0 = _s0_pass(src, dst, m2d.reshape(NPAD), zeros_s0)
        return (s0[:N, None], s0[NPAD:NPAD + N, None])

    s0a, s0b = s0_cols(ones2d)
    x8 = jnp.pad(x, ((0, 0), (0, 4)))
    w1p = jnp.pad(W1, ((0, 4), (0, 0)))
    hs, dm = _pre1(x8, s0a, s0b, w1p)
    s = _edge_pass128(src, dst, hs, zeros128)
    ones_col = jnp.ones((N, 1), f32)
    out, s_col = _post(s[:N], s[_NACC:_NACC + N], hs, dm, ones_col,
                       b1[None, :], p1[None, :])
    spad, mpad = _pad_scores(s_col, ones_col)
    m2d = _topk_kernels[5000](spad, mpad)
    m_col = m2d.reshape(NPAD)[:N, None]

    for W, b, p, k in ((W2, b2, p2, 2500), (W3, b3, p3, 1250),
                       (W4, b4, p4, 625)):
        s0a, s0b = s0_cols(m2d)
        hs, dm = _pre(out, s_col, m_col, s0a, s0b, W)
        s = _edge_pass128(src, dst, hs, zeros128)
        out, s_col = _post(s[:N], s[_NACC:_NACC + N], hs, dm, m_col,
                           b[None, :], p[None, :])
        spad, mpad = _pad_scores(s_col, m_col)
        m2d = _topk_kernels[k](spad, mpad)
        m_col = m2d.reshape(NPAD)[:N, None]

    meta128 = jnp.broadcast_to(metadata, (1, F))
    cw128 = jnp.pad(convm_w[None, :], ((0, 0), (0, F - 4)))
    cb128 = jnp.pad(convm_b[None, :], ((0, 0), (0, F - 4)))
    o = _head(out, s_col, m_col, meta128, cw128, cb128,
              fc_w, fc_b[None, :], fc2_w, fc2_b[None, :])
    return o.reshape((1,))
